# baseline (speedup 1.0000x reference)
import functools

import numpy as np
import jax
import jax.numpy as jnp
from jax import lax
from jax.experimental import pallas as pl
from jax.experimental.pallas import tpu as pltpu
from jax.experimental.pallas import tpu_sc as plsc

_W = 2048
_H = 2048
_SO = 0.6
_NP2 = 1 << 20
_ROWS = _NP2 // 128
_BR = 1024
_NC, _NS = 2, 16
_NW = _NC * _NS
_PER_W = _NP2 // _NW
_CHUNK = 2048
_NCHUNK = _PER_W // _CHUNK


def _prep_body(x_ref, y_ref, t_ref, il_ref, ir_ref, ic_ref):
    x = x_ref[...]
    y = y_ref[...]
    t = t_ref[...]
    for ang, out_ref in ((t - _SO, il_ref), (t + _SO, ir_ref), (t, ic_ref)):
        sx = x + jnp.cos(ang) * 1.0
        sy = y + jnp.sin(ang) * 1.0
        sx = jnp.maximum(0.0, jnp.minimum(sx, float(_W - 1)))
        sy = jnp.maximum(0.0, jnp.minimum(sy, float(_H - 1)))
        out_ref[...] = sx.astype(jnp.int32) * _H + sy.astype(jnp.int32)


_prep = pl.pallas_call(
    _prep_body,
    grid=(_ROWS // _BR,),
    in_specs=[pl.BlockSpec((_BR, 128), lambda i: (i, 0))] * 3,
    out_specs=[pl.BlockSpec((_BR, 128), lambda i: (i, 0))] * 3,
    out_shape=[jax.ShapeDtypeStruct((_ROWS, 128), jnp.int32)] * 3,
)


def _gather_body(table, il, ir, ic, ol, orr, oc, idx_v, val_v, sem):
    wid = lax.axis_index("s") * _NC + lax.axis_index("c")
    base = wid * _PER_W

    for idx_hbm, out_hbm in ((il, ol), (ir, orr), (ic, oc)):

        def body(c, _, idx_hbm=idx_hbm, out_hbm=out_hbm):
            off = pl.multiple_of(base + c * _CHUNK, _CHUNK)
            pltpu.sync_copy(idx_hbm.at[pl.ds(off, _CHUNK)], idx_v)
            pltpu.async_copy(table.at[idx_v], val_v, sem).wait()
            pltpu.sync_copy(val_v, out_hbm.at[pl.ds(off, _CHUNK)])
            return 0

        lax.fori_loop(0, _NCHUNK, body, 0)


_gather3 = functools.partial(
    pl.kernel,
    out_type=[jax.ShapeDtypeStruct((_NP2,), jnp.float32)] * 3,
    mesh=plsc.VectorSubcoreMesh(
        core_axis_name="c", subcore_axis_name="s", num_cores=_NC, num_subcores=_NS
    ),
    scratch_types=[
        pltpu.VMEM((_CHUNK,), jnp.int32),
        pltpu.VMEM((_CHUNK,), jnp.float32),
        pltpu.SemaphoreType.DMA,
    ],
)(_gather_body)


def _finish_body(
    x_ref, y_ref, t_ref, dl_ref, dr_ref, dc_ref, gl_ref, gr_ref, gc_ref, u_ref,
    xo_ref, yo_ref, to_ref,
):
    x = x_ref[...]
    y = y_ref[...]
    t = t_ref[...]
    dl = dl_ref[...]
    dr = dr_ref[...]
    dc = dc_ref[...]

    m = jnp.maximum(jnp.maximum(dl, dr), dc)
    el = jnp.exp(dl - m)
    er = jnp.exp(dr - m)
    ec = jnp.exp(dc - m)
    s = el + er + ec
    yl = el / s + gl_ref[...]
    yr = er / s + gr_ref[...]
    yc = ec / s + gc_ref[...]

    pick_l = (yl >= yr) & (yl >= yc)
    pick_r = jnp.logical_not(pick_l) & (yr >= yc)

    lx = jnp.cos(t - _SO)
    ly = jnp.sin(t - _SO)
    rx = jnp.cos(t + _SO)
    ry = jnp.sin(t + _SO)
    cx = jnp.cos(t)
    cy = jnp.sin(t)

    sx = jnp.where(pick_l, lx, jnp.where(pick_r, rx, cx))
    sy = jnp.where(pick_l, ly, jnp.where(pick_r, ry, cy))
    x1 = x + sx * 1.0
    y1 = y + sy * 1.0

    zeros = jnp.zeros_like(x1)
    ones = jnp.ones_like(x1)
    xcl = jnp.maximum(zeros, jnp.minimum(x1, ones * (_W - 1)))
    ycl = jnp.maximum(zeros, jnp.minimum(y1, ones * (_H - 1)))
    x_out = jnp.where(x1 >= _W, xcl, x1)
    x_out = jnp.where(x1 <= 0, xcl, x_out)
    y_out = jnp.where(y1 >= _H, ycl, y1)
    y_out = jnp.where(y1 <= 0, ycl, y_out)

    theta_rand = u_ref[...] * 2 * 3.141592
    cnt = jnp.where(x1 >= _W, ones, zeros)
    cnt = cnt + jnp.where(x1 <= 0, ones, zeros)
    cnt = cnt + jnp.where(y1 >= _H, ones, zeros)
    cnt = cnt + jnp.where(y1 <= 0, ones, zeros)
    t_out = cnt * theta_rand + jnp.abs(cnt - 1) * t

    xo_ref[...] = x_out
    yo_ref[...] = y_out
    to_ref[...] = t_out


_finish = pl.pallas_call(
    _finish_body,
    grid=(_ROWS // _BR,),
    in_specs=[pl.BlockSpec((_BR, 128), lambda i: (i, 0))] * 10,
    out_specs=[pl.BlockSpec((_BR, 128), lambda i: (i, 0))] * 3,
    out_shape=[jax.ShapeDtypeStruct((_ROWS, 128), jnp.float32)] * 3,
)


def _noise(n):
    skey = jax.random.key(42)
    ks, kt = jax.random.split(skey)
    g = jax.random.gumbel(ks, (n, 3), jnp.float32)
    u = jax.random.uniform(kt, (n,), jnp.float32)
    pad = _NP2 - n
    gl = jnp.pad(g[:, 0], (0, pad)).reshape(_ROWS, 128)
    gr = jnp.pad(g[:, 1], (0, pad)).reshape(_ROWS, 128)
    gc = jnp.pad(g[:, 2], (0, pad)).reshape(_ROWS, 128)
    up = jnp.pad(u, (0, pad)).reshape(_ROWS, 128)
    return gl, gr, gc, up


def kernel(x, y, theta, frame):
    n = x.shape[0]
    gl, gr, gc, u = _noise(n)
    pad = _NP2 - n
    xp = jnp.pad(x, (0, pad)).reshape(_ROWS, 128)
    yp = jnp.pad(y, (0, pad)).reshape(_ROWS, 128)
    tp = jnp.pad(theta, (0, pad)).reshape(_ROWS, 128)

    il, ir, ic = _prep(xp, yp, tp)

    table = frame.reshape(-1)
    dl, dr, dc = _gather3(
        table, il.reshape(-1), ir.reshape(-1), ic.reshape(-1)
    )

    xo, yo, to = _finish(
        xp, yp, tp,
        dl.reshape(_ROWS, 128), dr.reshape(_ROWS, 128), dc.reshape(_ROWS, 128),
        gl, gr, gc, u,
    )
    return (
        xo.reshape(_NP2)[:n],
        yo.reshape(_NP2)[:n],
        to.reshape(_NP2)[:n],
    )


# speedup vs baseline: 1.0955x; 1.0955x over previous
import functools

import numpy as np
import jax
import jax.numpy as jnp
from jax import lax
from jax.experimental import pallas as pl
from jax.experimental.pallas import tpu as pltpu
from jax.experimental.pallas import tpu_sc as plsc

_W = 2048
_H = 2048
_SO = 0.6
_NP2 = 1 << 20
_ROWS = _NP2 // 128
_BR = 1024
_NC, _NS = 2, 16
_NW = _NC * _NS
_PER_W = _NP2 // _NW
_CHUNK = 2048
_NCHUNK = _PER_W // _CHUNK


_C06 = np.float32(np.cos(0.6))
_S06 = np.float32(np.sin(0.6))


def _directions(ct, st):
    lx = ct * _C06 + st * _S06
    ly = st * _C06 - ct * _S06
    rx = ct * _C06 - st * _S06
    ry = st * _C06 + ct * _S06
    return lx, ly, rx, ry, ct, st


def _prep_body(x_ref, y_ref, t_ref, il_ref, ir_ref, ic_ref, ct_ref, st_ref):
    x = x_ref[...]
    y = y_ref[...]
    t = t_ref[...]
    ct = jnp.cos(t)
    st = jnp.sin(t)
    ct_ref[...] = ct
    st_ref[...] = st
    lx, ly, rx, ry, cx, cy = _directions(ct, st)
    for dx, dy, out_ref in ((lx, ly, il_ref), (rx, ry, ir_ref), (cx, cy, ic_ref)):
        sx = x + dx * 1.0
        sy = y + dy * 1.0
        sx = jnp.maximum(0.0, jnp.minimum(sx, float(_W - 1)))
        sy = jnp.maximum(0.0, jnp.minimum(sy, float(_H - 1)))
        out_ref[...] = sx.astype(jnp.int32) * _H + sy.astype(jnp.int32)


_prep = pl.pallas_call(
    _prep_body,
    grid=(_ROWS // _BR,),
    in_specs=[pl.BlockSpec((_BR, 128), lambda i: (i, 0))] * 3,
    out_specs=[pl.BlockSpec((_BR, 128), lambda i: (i, 0))] * 5,
    out_shape=[jax.ShapeDtypeStruct((_ROWS, 128), jnp.int32)] * 3
    + [jax.ShapeDtypeStruct((_ROWS, 128), jnp.float32)] * 2,
)


def _gather_body(table, il, ir, ic, ol, orr, oc, idx_v, val_v, sem):
    wid = lax.axis_index("s") * _NC + lax.axis_index("c")
    base = wid * _PER_W

    for idx_hbm, out_hbm in ((il, ol), (ir, orr), (ic, oc)):

        def body(c, _, idx_hbm=idx_hbm, out_hbm=out_hbm):
            off = pl.multiple_of(base + c * _CHUNK, _CHUNK)
            pltpu.sync_copy(idx_hbm.at[pl.ds(off, _CHUNK)], idx_v)
            pltpu.async_copy(table.at[idx_v], val_v, sem).wait()
            pltpu.sync_copy(val_v, out_hbm.at[pl.ds(off, _CHUNK)])
            return 0

        lax.fori_loop(0, _NCHUNK, body, 0)


_gather3 = functools.partial(
    pl.kernel,
    out_type=[jax.ShapeDtypeStruct((_NP2,), jnp.float32)] * 3,
    mesh=plsc.VectorSubcoreMesh(
        core_axis_name="c", subcore_axis_name="s", num_cores=_NC, num_subcores=_NS
    ),
    scratch_types=[
        pltpu.VMEM((_CHUNK,), jnp.int32),
        pltpu.VMEM((_CHUNK,), jnp.float32),
        pltpu.SemaphoreType.DMA,
    ],
)(_gather_body)


def _finish_body(
    x_ref, y_ref, t_ref, ct_ref, st_ref, dl_ref, dr_ref, dc_ref,
    gl_ref, gr_ref, gc_ref, u_ref,
    xo_ref, yo_ref, to_ref,
):
    x = x_ref[...]
    y = y_ref[...]
    t = t_ref[...]
    dl = dl_ref[...]
    dr = dr_ref[...]
    dc = dc_ref[...]

    m = jnp.maximum(jnp.maximum(dl, dr), dc)
    el = jnp.exp(dl - m)
    er = jnp.exp(dr - m)
    ec = jnp.exp(dc - m)
    s = el + er + ec
    yl = el / s + gl_ref[...]
    yr = er / s + gr_ref[...]
    yc = ec / s + gc_ref[...]

    pick_l = (yl >= yr) & (yl >= yc)
    pick_r = jnp.logical_not(pick_l) & (yr >= yc)

    lx, ly, rx, ry, cx, cy = _directions(ct_ref[...], st_ref[...])

    sx = jnp.where(pick_l, lx, jnp.where(pick_r, rx, cx))
    sy = jnp.where(pick_l, ly, jnp.where(pick_r, ry, cy))
    x1 = x + sx * 1.0
    y1 = y + sy * 1.0

    zeros = jnp.zeros_like(x1)
    ones = jnp.ones_like(x1)
    xcl = jnp.maximum(zeros, jnp.minimum(x1, ones * (_W - 1)))
    ycl = jnp.maximum(zeros, jnp.minimum(y1, ones * (_H - 1)))
    x_out = jnp.where(x1 >= _W, xcl, x1)
    x_out = jnp.where(x1 <= 0, xcl, x_out)
    y_out = jnp.where(y1 >= _H, ycl, y1)
    y_out = jnp.where(y1 <= 0, ycl, y_out)

    theta_rand = u_ref[...] * 2 * 3.141592
    cnt = jnp.where(x1 >= _W, ones, zeros)
    cnt = cnt + jnp.where(x1 <= 0, ones, zeros)
    cnt = cnt + jnp.where(y1 >= _H, ones, zeros)
    cnt = cnt + jnp.where(y1 <= 0, ones, zeros)
    t_out = cnt * theta_rand + jnp.abs(cnt - 1) * t

    xo_ref[...] = x_out
    yo_ref[...] = y_out
    to_ref[...] = t_out


_finish = pl.pallas_call(
    _finish_body,
    grid=(_ROWS // _BR,),
    in_specs=[pl.BlockSpec((_BR, 128), lambda i: (i, 0))] * 12,
    out_specs=[pl.BlockSpec((_BR, 128), lambda i: (i, 0))] * 3,
    out_shape=[jax.ShapeDtypeStruct((_ROWS, 128), jnp.float32)] * 3,
)


def _np_rotl(x, r):
    return ((x << np.uint32(r)) | (x >> np.uint32(32 - r))).astype(np.uint32)


def _np_threefry2x32(k1, k2, x0, x1):
    rot = (13, 15, 26, 6, 17, 29, 16, 24)
    ks0, ks1 = np.uint32(k1), np.uint32(k2)
    ks2 = np.uint32(ks0 ^ ks1 ^ np.uint32(0x1BD11BDA))
    x0 = (x0 + ks0).astype(np.uint32)
    x1 = (x1 + ks1).astype(np.uint32)
    keys = ((ks1, ks2), (ks2, ks0), (ks0, ks1), (ks1, ks2), (ks2, ks0))
    for i in range(5):
        rots = rot[:4] if i % 2 == 0 else rot[4:]
        for r in rots:
            x0 = (x0 + x1).astype(np.uint32)
            x1 = _np_rotl(x1, r)
            x1 = (x1 ^ x0).astype(np.uint32)
        a, b = keys[i]
        x0 = (x0 + a).astype(np.uint32)
        x1 = (x1 + b + np.uint32(i + 1)).astype(np.uint32)
    return x0, x1


def _np_bits(k1, k2, n):
    i = np.arange(n, dtype=np.uint64)
    hi = (i >> np.uint64(32)).astype(np.uint32)
    lo = (i & np.uint64(0xFFFFFFFF)).astype(np.uint32)
    b1, b2 = _np_threefry2x32(k1, k2, hi, lo)
    return (b1 ^ b2).astype(np.uint32)


def _np_uniform(k1, k2, n, minval=0.0, maxval=1.0):
    b = _np_bits(k1, k2, n)
    f = ((b >> np.uint32(9)) | np.uint32(0x3F800000)).view(np.float32)
    f = f - np.float32(1.0)
    out = f * np.float32(maxval - minval) + np.float32(minval)
    return np.maximum(np.float32(minval), out)


_NOISE_CACHE = {}


def _noise(n):
    if n not in _NOISE_CACHE:
        b1, b2 = _np_threefry2x32(
            np.uint32(0), np.uint32(42),
            np.zeros(2, np.uint32), np.arange(2, dtype=np.uint32),
        )
        ks = (b1[0], b2[0])
        kt = (b1[1], b2[1])
        tiny = float(np.finfo(np.float32).tiny)
        gu = _np_uniform(ks[0], ks[1], 3 * n, minval=tiny).reshape(n, 3)
        g = (-np.log(-np.log(gu))).astype(np.float32)
        u = _np_uniform(kt[0], kt[1], n)
        pad = _NP2 - n
        _NOISE_CACHE[n] = tuple(
            np.pad(a, (0, pad)).reshape(_ROWS, 128)
            for a in (g[:, 0], g[:, 1], g[:, 2], u)
        )
    return _NOISE_CACHE[n]


def kernel(x, y, theta, frame):
    n = x.shape[0]
    gl, gr, gc, u = _noise(n)
    pad = _NP2 - n
    xp = jnp.pad(x, (0, pad)).reshape(_ROWS, 128)
    yp = jnp.pad(y, (0, pad)).reshape(_ROWS, 128)
    tp = jnp.pad(theta, (0, pad)).reshape(_ROWS, 128)

    il, ir, ic, ct, st = _prep(xp, yp, tp)

    table = frame.reshape(-1)
    dl, dr, dc = _gather3(
        table, il.reshape(-1), ir.reshape(-1), ic.reshape(-1)
    )

    xo, yo, to = _finish(
        xp, yp, tp, ct, st,
        dl.reshape(_ROWS, 128), dr.reshape(_ROWS, 128), dc.reshape(_ROWS, 128),
        gl, gr, gc, u,
    )
    return (
        xo.reshape(_NP2)[:n],
        yo.reshape(_NP2)[:n],
        to.reshape(_NP2)[:n],
    )
